# baseline (speedup 1.0000x reference)
import functools

import jax
import jax.numpy as jnp
from jax import lax
from jax.experimental import pallas as pl
from jax.experimental.pallas import tpu as pltpu
from jax.experimental.pallas import tpu_sc as plsc

N_NODES = 10000
EMBED_DIM = 128
N_EDGES = 320000

NC = 2
NS = 16
NW = NC * NS

EPW = N_EDGES // NW
CHUNK = 128
NPH = 4
CPP = 20
NCH = NPH * CPP
PAD_ROW = N_NODES
ACC_ROWS = 10048
ZROWS = 64
NZ = ACC_ROWS // ZROWS


def _sc_aggregate(emb, dst3, src3, zrows):
    mesh = plsc.VectorSubcoreMesh(core_axis_name="c", subcore_axis_name="s")

    @functools.partial(
        pl.kernel,
        mesh=mesh,
        out_type=jax.ShapeDtypeStruct((NC, ACC_ROWS, EMBED_DIM), jnp.float32),
        scratch_types=[
            pltpu.VMEM((CPP, CHUNK), jnp.int32),
            pltpu.VMEM((CPP, CHUNK), jnp.int32),
            pltpu.VMEM((CHUNK, EMBED_DIM), jnp.float32),
            pltpu.VMEM((CHUNK, EMBED_DIM), jnp.float32),
            pltpu.VMEM_SHARED((ACC_ROWS, EMBED_DIM), jnp.float32),
            pltpu.SemaphoreType.DMA,
            pltpu.SemaphoreType.DMA,
        ],
    )
    def agg_kernel(emb_hbm, dst_hbm, src_hbm, z_hbm, out_hbm,
                   dst_v, src_v, rows0, rows1, acc, sem0, sem1):
        cid = lax.axis_index("c")
        sid = lax.axis_index("s")
        wid = cid * NS + sid

        pltpu.sync_copy(z_hbm, rows0.at[pl.ds(0, ZROWS)])

        def zero_body(k, carry):
            t = sid + NS * k

            @pl.when(t < NZ)
            def _():
                pltpu.sync_copy(rows0.at[pl.ds(0, ZROWS)],
                                acc.at[pl.ds(t * ZROWS, ZROWS)])

            return carry

        lax.fori_loop(0, (NZ + NS - 1) // NS, zero_body, 0)

        plsc.subcore_barrier()

        bufs = (rows0, rows1)
        sems = (sem0, sem1)
        for p in range(NPH):
            pltpu.sync_copy(dst_hbm.at[wid, p], dst_v)
            pltpu.sync_copy(src_hbm.at[wid, p], src_v)

            def body(g, carry):
                cps = [
                    pltpu.async_copy(
                        emb_hbm.at[src_v.at[2 * g + bf]], bufs[bf], sems[bf])
                    for bf in range(2)
                ]
                for bf in range(2):
                    cps[bf].wait()
                    pltpu.sync_copy(bufs[bf], acc.at[dst_v.at[2 * g + bf]],
                                    add=True)
                return carry

            lax.fori_loop(0, CPP // 2, body, 0)

        plsc.subcore_barrier()

        def out_body(k, carry):
            t = sid + NS * k

            @pl.when(t < NZ)
            def _():
                pltpu.sync_copy(acc.at[pl.ds(t * ZROWS, ZROWS)],
                                rows0.at[pl.ds(0, ZROWS)])
                pltpu.sync_copy(rows0.at[pl.ds(0, ZROWS)],
                                out_hbm.at[cid, pl.ds(t * ZROWS, ZROWS)])

            return carry

        lax.fori_loop(0, (NZ + NS - 1) // NS, out_body, 0)

    return agg_kernel(emb, dst3, src3, zrows)


def _tc_body(emb_ref, p_ref, wt_ref, b_ref, g_ref, bt_ref, o_ref):
    agg = p_ref[0] + p_ref[1]
    x = (
        jnp.dot(emb_ref[...], wt_ref[:EMBED_DIM, :],
                preferred_element_type=jnp.float32)
        + jnp.dot(agg, wt_ref[EMBED_DIM:, :],
                  preferred_element_type=jnp.float32)
        + b_ref[...]
    )
    mean = jnp.mean(x, axis=1, keepdims=True)
    xc = x - mean
    var = jnp.mean(xc * xc, axis=1, keepdims=True)
    o_ref[...] = (xc * lax.rsqrt(var + 1e-5)) * g_ref[...] + bt_ref[...]


def _tc_project(emb, partials, wt, b, gamma, beta):
    rows_per_block = 400
    grid = (N_NODES // rows_per_block,)
    return pl.pallas_call(
        _tc_body,
        grid=grid,
        in_specs=[
            pl.BlockSpec((rows_per_block, EMBED_DIM), lambda j: (j, 0)),
            pl.BlockSpec((NC, rows_per_block, EMBED_DIM), lambda j: (0, j, 0)),
            pl.BlockSpec((2 * EMBED_DIM, EMBED_DIM), lambda j: (0, 0)),
            pl.BlockSpec((1, EMBED_DIM), lambda j: (0, 0)),
            pl.BlockSpec((1, EMBED_DIM), lambda j: (0, 0)),
            pl.BlockSpec((1, EMBED_DIM), lambda j: (0, 0)),
        ],
        out_specs=pl.BlockSpec((rows_per_block, EMBED_DIM), lambda j: (j, 0)),
        out_shape=jax.ShapeDtypeStruct((N_NODES, EMBED_DIM), jnp.float32),
    )(emb, partials, wt, b, gamma, beta)


def kernel(emb, edges, W, b, ln_gamma, ln_beta):
    dst = edges[0].astype(jnp.int32).reshape(NW, EPW)
    src = edges[1].astype(jnp.int32).reshape(NW, EPW)
    pad = NCH * CHUNK - EPW
    dst3 = jnp.pad(dst, ((0, 0), (0, pad)),
                   constant_values=PAD_ROW).reshape(NW, NPH, CPP, CHUNK)
    src3 = jnp.pad(src, ((0, 0), (0, pad)),
                   constant_values=0).reshape(NW, NPH, CPP, CHUNK)
    zrows = jnp.zeros((ZROWS, EMBED_DIM), jnp.float32)

    partials = _sc_aggregate(emb, dst3, src3, zrows)

    wt = W.T
    return _tc_project(
        emb, partials, wt,
        b.reshape(1, EMBED_DIM),
        ln_gamma.reshape(1, EMBED_DIM),
        ln_beta.reshape(1, EMBED_DIM),
    )


# speedup vs baseline: 1.4289x; 1.4289x over previous
import functools

import jax
import jax.numpy as jnp
from jax import lax
from jax.experimental import pallas as pl
from jax.experimental.pallas import tpu as pltpu
from jax.experimental.pallas import tpu_sc as plsc

N_NODES = 10000
EMBED_DIM = 128
N_EDGES = 320000

NC = 2
NS = 16
NW = NC * NS

EPW = N_EDGES // NW
CHUNK = 128
NCH = (EPW + CHUNK - 1) // CHUNK
PAD_ROW = N_NODES
ACC_ROWS = 10112
RPS = ACC_ROWS // NS
ZROWS = 64


def _sc_aggregate(emb, dst3, src3, zrows):
    mesh = plsc.VectorSubcoreMesh(core_axis_name="c", subcore_axis_name="s")

    @functools.partial(
        pl.kernel,
        mesh=mesh,
        out_type=jax.ShapeDtypeStruct((NC, ACC_ROWS, EMBED_DIM), jnp.float32),
        scratch_types=[
            pltpu.VMEM((NCH, CHUNK), jnp.int32),
            pltpu.VMEM((NCH, CHUNK), jnp.int32),
            pltpu.VMEM((CHUNK, EMBED_DIM), jnp.float32),
            pltpu.VMEM((ZROWS, EMBED_DIM), jnp.float32),
            pltpu.VMEM_SHARED((ACC_ROWS, EMBED_DIM), jnp.float32),
            pltpu.SemaphoreType.DMA,
        ],
    )
    def agg_kernel(emb_hbm, dst_hbm, src_hbm, z_hbm, out_hbm,
                   dst_v, src_v, rows_v, zbuf, acc, sem):
        cid = lax.axis_index("c")
        sid = lax.axis_index("s")
        wid = cid * NS + sid
        rem = RPS - (RPS // ZROWS) * ZROWS

        pltpu.sync_copy(z_hbm, zbuf)

        def zero_body(t, carry):
            pltpu.sync_copy(zbuf, acc.at[pl.ds(sid * RPS + t * ZROWS, ZROWS)])
            return carry

        lax.fori_loop(0, RPS // ZROWS, zero_body, 0)
        pltpu.sync_copy(
            zbuf.at[pl.ds(0, rem)],
            acc.at[pl.ds(sid * RPS + RPS - rem, rem)])

        pltpu.sync_copy(dst_hbm.at[wid], dst_v)
        pltpu.sync_copy(src_hbm.at[wid], src_v)

        plsc.subcore_barrier()

        def body(j, carry):
            pltpu.async_copy(emb_hbm.at[src_v.at[j]], rows_v, sem).wait()
            pltpu.sync_copy(rows_v, acc.at[dst_v.at[j]], add=True)
            return carry

        lax.fori_loop(0, NCH, body, 0)

        plsc.subcore_barrier()

        def out_body(t, carry):
            r = sid * RPS + t * ZROWS
            pltpu.sync_copy(acc.at[pl.ds(r, ZROWS)], zbuf)
            pltpu.sync_copy(zbuf, out_hbm.at[cid, pl.ds(r, ZROWS)])
            return carry

        lax.fori_loop(0, RPS // ZROWS, out_body, 0)
        ro = sid * RPS + RPS - rem
        pltpu.sync_copy(acc.at[pl.ds(ro, rem)], zbuf.at[pl.ds(0, rem)])
        pltpu.sync_copy(zbuf.at[pl.ds(0, rem)], out_hbm.at[cid, pl.ds(ro, rem)])

    return agg_kernel(emb, dst3, src3, zrows)


def _tc_body(emb_ref, p_ref, wt_ref, b_ref, g_ref, bt_ref, o_ref):
    agg = p_ref[0] + p_ref[1]
    x = (
        jnp.dot(emb_ref[...], wt_ref[:EMBED_DIM, :],
                preferred_element_type=jnp.float32)
        + jnp.dot(agg, wt_ref[EMBED_DIM:, :],
                  preferred_element_type=jnp.float32)
        + b_ref[...]
    )
    mean = jnp.mean(x, axis=1, keepdims=True)
    xc = x - mean
    var = jnp.mean(xc * xc, axis=1, keepdims=True)
    o_ref[...] = (xc * lax.rsqrt(var + 1e-5)) * g_ref[...] + bt_ref[...]


def _tc_project(emb, partials, wt, b, gamma, beta):
    rows_per_block = 400
    grid = (N_NODES // rows_per_block,)
    return pl.pallas_call(
        _tc_body,
        grid=grid,
        in_specs=[
            pl.BlockSpec((rows_per_block, EMBED_DIM), lambda j: (j, 0)),
            pl.BlockSpec((NC, rows_per_block, EMBED_DIM), lambda j: (0, j, 0)),
            pl.BlockSpec((2 * EMBED_DIM, EMBED_DIM), lambda j: (0, 0)),
            pl.BlockSpec((1, EMBED_DIM), lambda j: (0, 0)),
            pl.BlockSpec((1, EMBED_DIM), lambda j: (0, 0)),
            pl.BlockSpec((1, EMBED_DIM), lambda j: (0, 0)),
        ],
        out_specs=pl.BlockSpec((rows_per_block, EMBED_DIM), lambda j: (j, 0)),
        out_shape=jax.ShapeDtypeStruct((N_NODES, EMBED_DIM), jnp.float32),
    )(emb, partials, wt, b, gamma, beta)


def kernel(emb, edges, W, b, ln_gamma, ln_beta):
    dst = edges[0].astype(jnp.int32).reshape(NW, EPW)
    src = edges[1].astype(jnp.int32).reshape(NW, EPW)
    pad = NCH * CHUNK - EPW
    dst3 = jnp.pad(dst, ((0, 0), (0, pad)),
                   constant_values=PAD_ROW).reshape(NW, NCH, CHUNK)
    src3 = jnp.pad(src, ((0, 0), (0, pad)),
                   constant_values=0).reshape(NW, NCH, CHUNK)
    zrows = jnp.zeros((ZROWS, EMBED_DIM), jnp.float32)

    partials = _sc_aggregate(emb, dst3, src3, zrows)

    wt = W.T
    return _tc_project(
        emb, partials, wt,
        b.reshape(1, EMBED_DIM),
        ln_gamma.reshape(1, EMBED_DIM),
        ln_beta.reshape(1, EMBED_DIM),
    )
